# Initial kernel scaffold; baseline (speedup 1.0000x reference)
import jax
import jax.numpy as jnp
from jax.experimental import pallas as pl


def kernel(content_feature, text_feature, cf_feature, bundle_idx, item_idx, c_W1, c_b1, c_W2, c_b2, c_W3, c_b3, t_W1, t_b1, t_W2, t_b2, t_W3, t_b3, cf_W, cf_b, item_embeddings, item_hyper_emb, Wq, Wk, Wv, modal_weight):
    raise NotImplementedError("write your pallas kernel here")


# speedup vs baseline: 6.0965x; 6.0965x over previous
import functools

import jax
import jax.numpy as jnp
from jax import lax
from jax.experimental import pallas as pl
from jax.experimental.pallas import tpu as pltpu
from jax.experimental.pallas import tpu_sc as plsc

N_ITEMS = 50000
N_BUNDLES = 10000
N_EDGES = 320000
D = 64
DA = 72

NC, NS = 2, 16
NW = NC * NS
CHUNK = 80

EPW = N_EDGES // NW
SB_STRIPE = 624
SB_REM = N_BUNDLES - NS * SB_STRIPE

IPC = N_ITEMS // NC
ROWS_PAD = IPC + 8
ROW_STRIPE = 1560
ZERO_REM = ROWS_PAD - NS * ROW_STRIPE
OUT_REM = IPC - NS * ROW_STRIPE
EPC = N_EDGES // NS


def _mesh():
    return plsc.VectorSubcoreMesh(core_axis_name="c", subcore_axis_name="s",
                                  num_cores=NC, num_subcores=NS)


_SC_PARAMS = pltpu.CompilerParams(use_tc_tiling_on_sc=False)


def _make_bundle_sum(width):
    def body(table, iidx, bidx, zrows, sb_out, acc, idxi_v, idxb_v, rows_v,
             sem):
        cid = lax.axis_index("c")
        sid = lax.axis_index("s")
        wid = cid * NS + sid
        pltpu.sync_copy(zrows.at[pl.ds(0, SB_STRIPE), :],
                        acc.at[pl.ds(sid * SB_STRIPE, SB_STRIPE), :])

        @pl.when(sid == 0)
        def _():
            pltpu.sync_copy(zrows.at[pl.ds(0, SB_REM), :],
                            acc.at[pl.ds(NS * SB_STRIPE, SB_REM), :])

        plsc.subcore_barrier()

        ebase = wid * EPW

        def step(i, carry):
            off = ebase + i * CHUNK
            pltpu.sync_copy(iidx.at[pl.ds(off, CHUNK)], idxi_v)
            pltpu.sync_copy(bidx.at[pl.ds(off, CHUNK)], idxb_v)
            pltpu.async_copy(table.at[idxi_v], rows_v, sem).wait()
            pltpu.sync_copy(rows_v, acc.at[idxb_v], add=True)
            return carry

        lax.fori_loop(0, EPW // CHUNK, step, 0)
        plsc.subcore_barrier()
        pltpu.sync_copy(acc.at[pl.ds(sid * SB_STRIPE, SB_STRIPE), :],
                        sb_out.at[cid, pl.ds(sid * SB_STRIPE, SB_STRIPE), :])

        @pl.when(sid == 0)
        def _():
            pltpu.sync_copy(acc.at[pl.ds(NS * SB_STRIPE, SB_REM), :],
                            sb_out.at[cid, pl.ds(NS * SB_STRIPE, SB_REM), :])

    return pl.kernel(
        body,
        out_type=jax.ShapeDtypeStruct((NC, N_BUNDLES, width), jnp.float32),
        mesh=_mesh(),
        compiler_params=_SC_PARAMS,
        scratch_types=[
            pltpu.VMEM_SHARED((N_BUNDLES, width), jnp.float32),
            pltpu.VMEM((CHUNK,), jnp.int32),
            pltpu.VMEM((CHUNK,), jnp.int32),
            pltpu.VMEM((CHUNK, width), jnp.float32),
            pltpu.SemaphoreType.DMA,
        ],
    )


@functools.cache
def _bundle_sum_aug():
    return _make_bundle_sum(DA)


@functools.cache
def _bundle_sum_plain():
    return _make_bundle_sum(D)


def _item_sum_body(table, iidx, bidx, zrows, si_out,
                   acc, idxb_v, idxl_v, rows_v, sem):
    cid = lax.axis_index("c")
    sid = lax.axis_index("s")
    pltpu.sync_copy(zrows.at[pl.ds(0, ROW_STRIPE), :],
                    acc.at[pl.ds(sid * ROW_STRIPE, ROW_STRIPE), :])

    @pl.when(sid == 0)
    def _():
        pltpu.sync_copy(zrows.at[pl.ds(0, ZERO_REM), :],
                        acc.at[pl.ds(NS * ROW_STRIPE, ZERO_REM), :])

    plsc.subcore_barrier()

    item_base = cid * IPC
    ebase = sid * EPC

    def step(i, carry):
        off = ebase + i * CHUNK
        pltpu.sync_copy(bidx.at[pl.ds(off, CHUNK)], idxb_v)
        pltpu.sync_copy(iidx.at[pl.ds(off, CHUNK)], idxl_v)
        for j in range(CHUNK // 16):
            v = idxl_v[pl.ds(j * 16, 16)]
            lo = v - item_base
            ok = (lo >= 0) & (lo < IPC)
            idxl_v[pl.ds(j * 16, 16)] = jnp.where(ok, lo, IPC)
        pltpu.async_copy(table.at[idxb_v], rows_v, sem).wait()
        pltpu.sync_copy(rows_v, acc.at[idxl_v], add=True)
        return carry

    lax.fori_loop(0, EPC // CHUNK, step, 0)
    plsc.subcore_barrier()
    pltpu.sync_copy(acc.at[pl.ds(sid * ROW_STRIPE, ROW_STRIPE), :],
                    si_out.at[pl.ds(item_base + sid * ROW_STRIPE,
                                    ROW_STRIPE), :])

    @pl.when(sid == 0)
    def _():
        pltpu.sync_copy(acc.at[pl.ds(NS * ROW_STRIPE, OUT_REM), :],
                        si_out.at[pl.ds(item_base + NS * ROW_STRIPE,
                                        OUT_REM), :])


@functools.cache
def _item_sum():
    return pl.kernel(
        _item_sum_body,
        out_type=jax.ShapeDtypeStruct((N_ITEMS, DA), jnp.float32),
        mesh=_mesh(),
        compiler_params=_SC_PARAMS,
        scratch_types=[
            pltpu.VMEM_SHARED((ROWS_PAD, DA), jnp.float32),
            pltpu.VMEM((CHUNK,), jnp.int32),
            pltpu.VMEM((CHUNK,), jnp.int32),
            pltpu.VMEM((CHUNK, DA), jnp.float32),
            pltpu.SemaphoreType.DMA,
        ],
    )


def _inv_deg(deg):
    return jnp.where(deg > 0, 1.0 / jnp.maximum(deg, 1e-8), 0.0)


def _make_y_body(s0_ref, s1_ref, y_ref):
    s = s0_ref[...] + s1_ref[...]
    inv = _inv_deg(s[:, D:D + 1])
    y_ref[...] = jnp.concatenate(
        [s[:, :D] * inv,
         jnp.ones((N_BUNDLES, 1), jnp.float32),
         jnp.zeros((N_BUNDLES, DA - D - 1), jnp.float32)], axis=1)


_make_y = pl.pallas_call(
    _make_y_body,
    out_shape=jax.ShapeDtypeStruct((N_BUNDLES, DA), jnp.float32),
)


def _final_body(s0_ref, s1_ref, d0_ref, d1_ref, o_ref):
    deg = d0_ref[...] + d1_ref[...]
    o_ref[...] = (s0_ref[...] + s1_ref[...]) * _inv_deg(deg)


_final_scale = pl.pallas_call(
    _final_body,
    out_shape=jax.ShapeDtypeStruct((N_BUNDLES, D), jnp.float32),
)


def _l2norm(x):
    n = jnp.sqrt(jnp.sum(x * x, axis=1, keepdims=True))
    return x / jnp.maximum(n, 1e-12)


def _fused_body(content_ref, text_ref, cf_ref, ie_ref, h0_ref, si_ref,
                cw1, cb1, cw2, cb2, cw3, cb3,
                tw1, tb1, tw2, tb2, tw3, tb3,
                cfw, cfb, wq, wk, wv, out_ref):
    f32 = jnp.float32

    def dense3(x, w1, b1, w2, b2, w3, b3):
        h = jnp.dot(x, w1[...], preferred_element_type=f32) + b1[...][None, :]
        h = jnp.maximum(h, 0.0)
        h = jnp.dot(h, w2[...], preferred_element_type=f32) + b2[...][None, :]
        h = jnp.maximum(h, 0.0)
        return jnp.dot(h, w3[...], preferred_element_type=f32) + b3[...][None, :]

    c = dense3(_l2norm(content_ref[...]), cw1, cb1, cw2, cb2, cw3, cb3)
    t = dense3(_l2norm(text_ref[...]), tw1, tb1, tw2, tb2, tw3, tb3)
    mm = c + t
    cfo = (jnp.dot(cf_ref[...], cfw[...], preferred_element_type=f32)
           + cfb[...][None, :])
    sia = si_ref[...]
    h1 = sia[:, :D] * _inv_deg(sia[:, D:D + 1])
    hyper = _l2norm(0.5 * (h0_ref[...] + h1))

    views = [mm, cfo, ie_ref[...], hyper]
    fs = []
    for v in views:
        mu = jnp.mean(v, axis=1, keepdims=True)
        d = v - mu
        var = jnp.mean(d * d, axis=1, keepdims=True)
        fs.append(d * lax.rsqrt(var + 1e-5))
    bsz = fs[0].shape[0]
    F = jnp.concatenate(fs, axis=0)
    Q = jnp.dot(F, wq[...], preferred_element_type=f32)
    K = jnp.dot(F, wk[...], preferred_element_type=f32)
    V = jnp.dot(F, wv[...], preferred_element_type=f32)
    qs = [Q[i * bsz:(i + 1) * bsz] for i in range(4)]
    ks = [K[i * bsz:(i + 1) * bsz] for i in range(4)]
    vs = [V[i * bsz:(i + 1) * bsz] for i in range(4)]
    scale = float(D) ** -0.5
    acc = jnp.zeros_like(vs[0])
    for i in range(4):
        s = [jnp.sum(qs[i] * ks[j], axis=1, keepdims=True) * scale
             for j in range(4)]
        m = jnp.maximum(jnp.maximum(s[0], s[1]), jnp.maximum(s[2], s[3]))
        e = [jnp.exp(sj - m) for sj in s]
        den = e[0] + e[1] + e[2] + e[3]
        o = (e[0] * vs[0] + e[1] * vs[1] + e[2] * vs[2] + e[3] * vs[3]) / den
        acc = acc + o
    out_ref[...] = 0.25 * acc


_BI = 1000
_GRID = N_ITEMS // _BI


def _row_spec(dcols):
    return pl.BlockSpec((_BI, dcols), lambda i: (i, 0))


def _full_spec(shape):
    nd = len(shape)
    return pl.BlockSpec(shape, (lambda i: (0,) * nd))


_fused_items_specs = [
    _row_spec(512), _row_spec(768), _row_spec(D), _row_spec(D),
    _row_spec(D), _row_spec(DA),
    _full_spec((512, 512)), _full_spec((512,)),
    _full_spec((512, 256)), _full_spec((256,)),
    _full_spec((256, D)), _full_spec((D,)),
    _full_spec((768, 768)), _full_spec((768,)),
    _full_spec((768, 256)), _full_spec((256,)),
    _full_spec((256, D)), _full_spec((D,)),
    _full_spec((D, D)), _full_spec((D,)),
    _full_spec((D, D)), _full_spec((D, D)), _full_spec((D, D)),
]

_fused_items = pl.pallas_call(
    _fused_body,
    grid=(_GRID,),
    in_specs=_fused_items_specs,
    out_specs=_row_spec(D),
    out_shape=jax.ShapeDtypeStruct((N_ITEMS, D), jnp.float32),
)


def kernel(content_feature, text_feature, cf_feature, bundle_idx, item_idx,
           c_W1, c_b1, c_W2, c_b2, c_W3, c_b3,
           t_W1, t_b1, t_W2, t_b2, t_W3, t_b3,
           cf_W, cf_b, item_embeddings, item_hyper_emb,
           Wq, Wk, Wv, modal_weight):
    wm = jax.nn.softmax(modal_weight, axis=0)
    zrows_a = jnp.zeros((ROW_STRIPE + ZERO_REM, DA), jnp.float32)
    zrows_p = jnp.zeros((SB_STRIPE + SB_REM, D), jnp.float32)

    h0_aug = jnp.concatenate(
        [item_hyper_emb,
         jnp.ones((N_ITEMS, 1), jnp.float32),
         jnp.zeros((N_ITEMS, DA - D - 1), jnp.float32)], axis=1)
    sb_part = _bundle_sum_aug()(h0_aug, item_idx, bundle_idx, zrows_a)
    y_aug = _make_y(sb_part[0], sb_part[1])

    si_aug = _item_sum()(y_aug, item_idx, bundle_idx, zrows_a)

    item_feature = _fused_items(
        content_feature, text_feature, cf_feature, item_embeddings,
        item_hyper_emb, si_aug,
        c_W1, c_b1, c_W2, c_b2, wm[0] * c_W3, wm[0] * c_b3,
        t_W1, t_b1, t_W2, t_b2, wm[1] * t_W3, wm[1] * t_b3,
        cf_W, cf_b, Wq, Wk, Wv)

    sbf_part = _bundle_sum_plain()(item_feature, item_idx, bundle_idx,
                                   zrows_p)
    db0 = sb_part[0, :, D:D + 1]
    db1 = sb_part[1, :, D:D + 1]
    return _final_scale(sbf_part[0], sbf_part[1], db0, db1)
